# baseline (speedup 1.0000x reference)
import jax
import jax.numpy as jnp
from jax import lax
from jax.experimental import pallas as pl
from jax.experimental.pallas import tpu as pltpu
from jax.experimental.pallas import tpu_sc as plsc

_START_GENE = 65000
_B = 16384
_D = 64

_info = plsc.get_sparse_core_info()
_NC = _info.num_cores
_NS = _info.num_subcores
_NW = _NC * _NS

_COL0 = _START_GENE // 128
_CPW = 245
_SPAN = _CPW * 128
_BLK = 128
_NSR = 16
_BPS = 16
_SRROWS = _BPS * _BLK
_NBLK = _NSR * _BPS
_RING = 4
_RND = 512
_LW = _RND + 48
_LTRASH = _RND + 32
_ICHUNK = 2048


def _gather_body(table_t, idx_hbm, out_hbm,
                 idx_v, gr_l, pr_l, subg, subpos, msr_v, pos2d,
                 bufs, rows_buf, sc16, sems):
    wid = lax.axis_index("s") * _NC + lax.axis_index("c")
    lo = (_COL0 * 128) + _SPAN * wid
    hi = lo + _SPAN
    col0 = lo // 128

    iota16 = lax.broadcasted_iota(jnp.int32, (16,), 0)
    zeros16 = jnp.zeros((16,), jnp.int32)

    def prefix_sum16(m):
        s = m
        for k in (1, 2, 4, 8):
            sc16[pl.ds(0, 16)] = s
            shifted = plsc.load_gather(sc16, [jnp.maximum(iota16 - k, 0)])
            s = s + jnp.where(iota16 >= k, shifted, 0)
        return s

    acc = zeros16
    for ci in range(_B // _ICHUNK):
        pltpu.sync_copy(idx_hbm.at[pl.ds(ci * _ICHUNK, _ICHUNK)], idx_v)

        def count_group(gi, a):
            g_vec = idx_v[pl.ds(gi * 16, 16)] + _START_GENE
            return a + jnp.where((g_vec >= lo) & (g_vec < hi), 1, 0)

        acc = lax.fori_loop(0, _ICHUNK // 16, count_group, acc)
    n_w = prefix_sum16(acc)[15]

    def round_body(r, carry):
        del carry
        pbase = r * _RND
        n_round = jnp.minimum(n_w - pbase, _RND)
        tr = (n_round + 15) // 16

        def scan_chunk(ci, n):
            pltpu.sync_copy(idx_hbm.at[pl.ds(ci * _ICHUNK, _ICHUNK)], idx_v)

            def scan_group(gi, n2):
                g_vec = idx_v[pl.ds(gi * 16, 16)] + _START_GENE
                pos_vec = zeros16 + ci * _ICHUNK + gi * 16 + iota16
                mask_b = (g_vec >= lo) & (g_vec < hi)
                cum = prefix_sum16(jnp.where(mask_b, 1, 0))
                slot = n2 + cum - 1
                mask_w = mask_b & (slot >= pbase) & (slot < pbase + _RND)
                tgt = jnp.where(mask_w, slot - pbase, _LTRASH)
                plsc.store_scatter(gr_l, [tgt], g_vec)
                plsc.store_scatter(pr_l, [tgt], pos_vec)
                return n2 + cum[15]

            return lax.fori_loop(0, _ICHUNK // 16, scan_group, n)

        lax.fori_loop(0, _B // _ICHUNK, scan_chunk, jnp.int32(0))

        p0vec = zeros16 + pr_l[pl.ds(0, 16)][0]
        for t in range(_RND // 16):
            lane = zeros16 + t * 16 + iota16
            row, start = t // 8, (t % 8) * 16
            v = pr_l[pl.ds(t * 16, 16)]
            pos2d[row, pl.ds(start, 16)] = jnp.where(lane < n_round, v, p0vec)

        def bucket_sr(sr, carry2):
            srlo = lo + sr * _SRROWS

            def sr_group(t, ms):
                g_vec = gr_l[pl.ds(t * 16, 16)]
                slot_vec = zeros16 + t * 16 + iota16
                valid = slot_vec < n_round
                mask_b = (g_vec >= srlo) & (g_vec < srlo + _SRROWS) & valid
                cum = prefix_sum16(jnp.where(mask_b, 1, 0))
                tgt = jnp.where(mask_b, sr * _LW + ms + cum - 1,
                                sr * _LW + _LTRASH)
                plsc.store_scatter(subg, [tgt], g_vec)
                plsc.store_scatter(subpos, [tgt], slot_vec)
                return ms + cum[15]

            m_sr = lax.fori_loop(0, tr, sr_group, jnp.int32(0))
            plsc.store_scatter(msr_v, [zeros16 + sr], zeros16 + m_sr)
            return carry2

        lax.fori_loop(0, _NSR, bucket_sr, 0)

        def start_blk(q, col):
            coff = pl.multiple_of(col * 128, 128)
            for tj in range(_D // 8):
                pltpu.async_copy(
                    table_t.at[pl.ds(8 * tj, 8), pl.ds(coff, _BLK)],
                    bufs[q].at[pl.ds(8 * tj, 8)], sems[q])

        def wait_blk(q):
            pltpu.make_async_copy(
                table_t.at[:, pl.ds(0, _BLK)], bufs[q], sems[q]).wait()

        def process_block(q, bg):
            sr = bg >> 4
            m_sr = plsc.load_gather(msr_v, [zeros16 + sr])[0]
            cb = (col0 + bg) * 128
            base = sr * _LW

            def match_group(t, carry2):
                g_vec = subg[pl.ds(base + t * 16, 16)]
                slot_vec = subpos[pl.ds(base + t * 16, 16)]
                valid = (zeros16 + t * 16 + iota16) < m_sr
                mask_i = jnp.where(
                    (g_vec >= cb) & (g_vec < cb + _BLK) & valid, 1, 0)
                for j in range(16):
                    @pl.when(mask_i[j] > 0)
                    def _():
                        cvec = zeros16 + (g_vec[j] - cb)
                        svec = zeros16 + slot_vec[j]
                        for m in range(_D // 16):
                            fid = iota16 + 16 * m
                            vals = plsc.load_gather(bufs[q], [fid, cvec])
                            plsc.store_scatter(rows_buf, [svec, fid], vals)
                return carry2

            lax.fori_loop(0, (m_sr + 15) // 16, match_group, 0)

        for q in range(_RING):
            start_blk(q, col0 + q)

        def ring_body(ib, carry2):
            for q in range(_RING):
                bg = ib * _RING + q
                wait_blk(q)
                process_block(q, bg)
                start_blk(q, col0 + bg + _RING)
            return carry2

        lax.fori_loop(0, _NBLK // _RING, ring_body, 0)
        for q in range(_RING):
            wait_blk(q)

        r0 = [rows_buf[0, pl.ds(16 * m, 16)] for m in range(_D // 16)]

        def pad_rows(s2, carry2):
            svec = zeros16 + s2
            for m in range(_D // 16):
                plsc.store_scatter(rows_buf, [svec, iota16 + 16 * m], r0[m])
            return carry2

        lax.fori_loop(n_round, _RND, pad_rows, 0)

        def scatter_chunk(c, carry2):
            @pl.when(c * 128 < n_round)
            def _():
                pltpu.sync_copy(
                    rows_buf.at[pl.ds(c * 128, 128)],
                    out_hbm.at[pos2d.at[c]],
                )
            return carry2

        lax.fori_loop(0, _RND // 128, scatter_chunk, 0)
        return 0

    nrounds = (n_w + _RND - 1) // _RND
    lax.fori_loop(0, nrounds, round_body, 0)


@jax.jit
def kernel(embedding_weight, batch):
    idx = batch.astype(jnp.int32)
    mesh = plsc.VectorSubcoreMesh(core_axis_name="c", subcore_axis_name="s")

    def body(table_t, idx_hbm, out_hbm, idx_v, gr_l, pr_l, subg, subpos,
             msr_v, pos2d, b0, b1, b2, b3, rows_buf, sc16,
             s0, s1, s2, s3):
        _gather_body(table_t, idx_hbm, out_hbm, idx_v, gr_l, pr_l, subg,
                     subpos, msr_v, pos2d, [b0, b1, b2, b3],
                     rows_buf, sc16, [s0, s1, s2, s3])

    return pl.kernel(
        body,
        mesh=mesh,
        compiler_params=pltpu.CompilerParams(needs_layout_passes=False),
        out_type=jax.ShapeDtypeStruct((_B, 128), jnp.float32),
        scratch_types=[
            pltpu.VMEM((_ICHUNK,), jnp.int32),
            pltpu.VMEM((_LW,), jnp.int32),
            pltpu.VMEM((_LW,), jnp.int32),
            pltpu.VMEM((_NSR * _LW,), jnp.int32),
            pltpu.VMEM((_NSR * _LW,), jnp.int32),
            pltpu.VMEM((16,), jnp.int32),
            pltpu.VMEM((_RND // 128 + 1, 128), jnp.int32),
            pltpu.VMEM((_D, _BLK), jnp.float32),
            pltpu.VMEM((_D, _BLK), jnp.float32),
            pltpu.VMEM((_D, _BLK), jnp.float32),
            pltpu.VMEM((_D, _BLK), jnp.float32),
            pltpu.VMEM((_RND + 16, 128), jnp.float32),
            pltpu.VMEM((16,), jnp.int32),
            pltpu.SemaphoreType.DMA,
            pltpu.SemaphoreType.DMA,
            pltpu.SemaphoreType.DMA,
            pltpu.SemaphoreType.DMA,
        ],
    )(embedding_weight.T, idx)[:, :_D]


# speedup vs baseline: 1.1235x; 1.1235x over previous
import jax
import jax.numpy as jnp
from jax import lax
from jax.experimental import pallas as pl
from jax.experimental.pallas import tpu as pltpu
from jax.experimental.pallas import tpu_sc as plsc

_START_GENE = 65000
_B = 16384
_D = 64

_info = plsc.get_sparse_core_info()
_NC = _info.num_cores
_NS = _info.num_subcores
_NW = _NC * _NS
_BPW = _B // _NW


def _gather_body(table_hbm, idx_hbm, out_hbm, idx_v, idx_s, rows_v, sem):
    wid = lax.axis_index("s") * _NC + lax.axis_index("c")
    base = wid * _BPW
    del idx_s
    pltpu.sync_copy(idx_hbm.at[pl.ds(base, _BPW)], idx_v)

    def issue_group(g, carry):
        vec = idx_v[pl.ds(g * 16, 16)] + _START_GENE
        for j in range(16):
            r = vec[j]
            pltpu.async_copy(
                table_hbm.at[pl.ds(r, 1)], rows_v.at[pl.ds(g * 16 + j, 1)], sem
            )
        return carry

    lax.fori_loop(0, _BPW // 16, issue_group, 0)

    pltpu.make_async_copy(table_hbm.at[pl.ds(0, _BPW)], rows_v, sem).wait()

    pltpu.sync_copy(rows_v, out_hbm.at[pl.ds(base, _BPW)])


@jax.jit
def kernel(embedding_weight, batch):
    idx = batch.astype(jnp.int32)
    mesh = plsc.VectorSubcoreMesh(core_axis_name="c", subcore_axis_name="s")
    return pl.kernel(
        _gather_body,
        mesh=mesh,
        out_type=jax.ShapeDtypeStruct((_B, _D), jnp.float32),
        scratch_types=[
            pltpu.VMEM((_BPW,), jnp.int32),
            pltpu.SMEM((_BPW,), jnp.int32),
            pltpu.VMEM((_BPW, _D), jnp.float32),
            pltpu.SemaphoreType.DMA,
        ],
    )(embedding_weight, idx)
